# Initial kernel scaffold; baseline (speedup 1.0000x reference)
import jax
import jax.numpy as jnp
from jax.experimental import pallas as pl


def kernel(x, up_index, boundary_index, boundary_attr, params):
    raise NotImplementedError("write your pallas kernel here")


# speedup vs baseline: 3.8468x; 3.8468x over previous
import functools

import jax
import jax.numpy as jnp
from jax import lax
from jax.experimental import pallas as pl
from jax.experimental.pallas import tpu as pltpu
from jax.experimental.pallas import tpu_sc as plsc

H = 64
HH = 32
NC = 2
NS = 16
KI = 128
CH = 1024
NPAD = 50048
EPAD = 819200
ZCH = 136
CH2 = 512
BR = 3128


def _pre_body(n_real, x_ref, ba_ref, wmu_ref, bmu_ref, wmb_ref, bmb_ref,
              au_ref, bu_ref, ab_ref, bb_ref):
    i = pl.program_id(0)
    rows = i * BR + lax.broadcasted_iota(jnp.int32, (BR, 1), 0)
    valid = (rows < n_real).astype(jnp.float32)
    x = x_ref[...] * valid
    ba = ba_ref[...] * valid
    wmu = wmu_ref[...]
    wmb = wmb_ref[...]
    f32 = jnp.float32
    au_ref[0] = (jnp.dot(x, wmu[:H, :HH], preferred_element_type=f32)
                 + bmu_ref[0, :HH]) * valid
    au_ref[1] = (jnp.dot(x, wmu[:H, HH:], preferred_element_type=f32)
                 + bmu_ref[0, HH:]) * valid
    bu_ref[0] = jnp.dot(x, wmu[H:, :HH], preferred_element_type=f32)
    bu_ref[1] = jnp.dot(x, wmu[H:, HH:], preferred_element_type=f32)
    ab_ref[0] = (jnp.dot(x, wmb[:H, :HH], preferred_element_type=f32)
                 + bmb_ref[0, :HH]) * valid
    ab_ref[1] = (jnp.dot(x, wmb[:H, HH:], preferred_element_type=f32)
                 + bmb_ref[0, HH:]) * valid
    bb_ref[0] = jnp.dot(ba, wmb[H:, :HH], preferred_element_type=f32)
    bb_ref[1] = jnp.dot(ba, wmb[H:, HH:], preferred_element_type=f32)


def _make_tables(x, ba, wmu, bmu, wmb, bmb, n_real):
    grid = NPAD // BR
    tab = jax.ShapeDtypeStruct((NC, NPAD, HH), jnp.float32)
    in_specs = [
        pl.BlockSpec((BR, H), lambda i: (i, 0)),
        pl.BlockSpec((BR, H), lambda i: (i, 0)),
        pl.BlockSpec((2 * H, H), lambda i: (0, 0)),
        pl.BlockSpec((1, H), lambda i: (0, 0)),
        pl.BlockSpec((2 * H, H), lambda i: (0, 0)),
        pl.BlockSpec((1, H), lambda i: (0, 0)),
    ]
    out_spec = pl.BlockSpec((NC, BR, HH), lambda i: (0, i, 0))
    return pl.pallas_call(
        functools.partial(_pre_body, n_real),
        grid=(grid,),
        in_specs=in_specs,
        out_specs=[out_spec] * 4,
        out_shape=[tab] * 4,
    )(x, ba, wmu, bmu, wmb, bmb)


def _pass1_body(a_hbm, b_hbm, ti_hbm, si_hbm, h_hbm, mom_hbm,
                tiv, siv, av, bv, momv, sem):
    c = lax.axis_index("c")
    s = lax.axis_index("s")
    e_pc = EPAD // NS
    nchunks = e_pc // CH
    nsub = CH // KI
    a_tab = a_hbm.at[c]
    b_tab = b_hbm.at[c]
    h_out = h_hbm.at[c]
    row0 = s * (e_pc // KI)

    def chunk(j, carry):
        s0, s1, q0, q1 = carry
        r = row0 + j * nsub
        pltpu.sync_copy(ti_hbm.at[pl.ds(r, nsub)], tiv)
        pltpu.sync_copy(si_hbm.at[pl.ds(r, nsub)], siv)
        descs = []
        for k in range(nsub):
            descs.append(pltpu.async_copy(
                a_tab.at[tiv.at[k]], av.at[pl.ds(k * KI, KI)], sem))
            descs.append(pltpu.async_copy(
                b_tab.at[siv.at[k]], bv.at[pl.ds(k * KI, KI)], sem))
        for d in descs:
            d.wait()

        def rowfn(i, cr):
            t0, t1, p0, p1 = cr
            h0 = av[i, pl.ds(0, 16)] + bv[i, pl.ds(0, 16)]
            av[i, pl.ds(0, 16)] = h0
            h1 = av[i, pl.ds(16, 16)] + bv[i, pl.ds(16, 16)]
            av[i, pl.ds(16, 16)] = h1
            return (t0 + h0, t1 + h1, p0 + h0 * h0, p1 + h1 * h1)

        carry2 = lax.fori_loop(0, CH, rowfn, (s0, s1, q0, q1))
        pltpu.sync_copy(av, h_out.at[pl.ds(s * e_pc + j * CH, CH)])
        return carry2

    z = jnp.zeros((16,), jnp.float32)
    s0, s1, q0, q1 = lax.fori_loop(0, nchunks, chunk, (z, z, z, z))
    momv[0, pl.ds(0, 16)] = s0
    momv[0, pl.ds(16, 16)] = s1
    momv[1, pl.ds(0, 16)] = q0
    momv[1, pl.ds(16, 16)] = q1
    pltpu.sync_copy(momv, mom_hbm.at[c].at[s])


def _pass1(a_tab, b_tab, ti, si):
    mesh = plsc.VectorSubcoreMesh(
        core_axis_name="c", subcore_axis_name="s",
        num_cores=NC, num_subcores=NS)
    nsub = CH // KI
    return pl.kernel(
        _pass1_body,
        compiler_params=pltpu.CompilerParams(use_tc_tiling_on_sc=False),
        out_type=(
            jax.ShapeDtypeStruct((NC, EPAD, HH), jnp.float32),
            jax.ShapeDtypeStruct((NC, NS, 2, HH), jnp.float32),
        ),
        mesh=mesh,
        scratch_types=[
            pltpu.VMEM((nsub, KI), jnp.int32),
            pltpu.VMEM((nsub, KI), jnp.int32),
            pltpu.VMEM((CH, HH), jnp.float32),
            pltpu.VMEM((CH, HH), jnp.float32),
            pltpu.VMEM((2, HH), jnp.float32),
            pltpu.SemaphoreType.DMA,
        ],
    )(a_tab, b_tab, ti, si)


def _mid_body(mu_ref, mb_ref, gu_ref, beu_ref, gb_ref, beb_ref, e_ref,
              ssu_ref, ssb_ref):
    e = e_ref[0, 0]

    def one(m_ref, g_ref, be_ref, out_ref):
        m = m_ref[...]
        tot = jnp.sum(m, axis=1)
        mu = tot[:, 0, :] / e
        var = tot[:, 1, :] / e - mu * mu
        scale = g_ref[...] * jax.lax.rsqrt(var + 1e-5)
        shift = be_ref[...] - mu * scale
        out_ref[:, 0, :] = scale
        out_ref[:, 1, :] = shift

    one(mu_ref, gu_ref, beu_ref, ssu_ref)
    one(mb_ref, gb_ref, beb_ref, ssb_ref)


def _mid(mom_u, mom_b, gu, beu, gb, beb, e_total):
    e = jnp.full((1, 1), float(e_total), jnp.float32)
    ss = jax.ShapeDtypeStruct((NC, 2, HH), jnp.float32)
    return pl.pallas_call(_mid_body, out_shape=(ss, ss))(
        mom_u, mom_b, gu, beu, gb, beb, e)


def _pass2_body(h_hbm, ti_hbm, ss_hbm, agg_hbm,
                shared, tiv, yv, ssv, zv, sem):
    c = lax.axis_index("c")
    s = lax.axis_index("s")
    e_pc = EPAD // NS
    nchunks = e_pc // CH
    nsub = CH // KI
    rows_pt = NPAD // NS

    pltpu.sync_copy(ss_hbm.at[c], ssv)

    def zrow(i, _):
        zv[i, pl.ds(0, 16)] = jnp.zeros((16,), jnp.float32)
        zv[i, pl.ds(16, 16)] = jnp.zeros((16,), jnp.float32)
        return 0

    lax.fori_loop(0, ZCH, zrow, 0)
    for t in range(rows_pt // ZCH):
        pltpu.sync_copy(zv, shared.at[pl.ds(s * rows_pt + t * ZCH, ZCH)])
    plsc.subcore_barrier()

    sc0 = ssv[0, pl.ds(0, 16)]
    sc1 = ssv[0, pl.ds(16, 16)]
    sh0 = ssv[1, pl.ds(0, 16)]
    sh1 = ssv[1, pl.ds(16, 16)]
    h_in = h_hbm.at[c]
    row0 = s * (e_pc // KI)
    nsub2 = CH2 // KI

    def chunk(j, _):
        r = row0 + j * nsub
        pltpu.sync_copy(ti_hbm.at[pl.ds(r, nsub)], tiv)
        for half in range(CH // CH2):
            pltpu.sync_copy(
                h_in.at[pl.ds(s * e_pc + j * CH + half * CH2, CH2)], yv)

            def rowfn(i, _2):
                y0 = yv[i, pl.ds(0, 16)] * sc0 + sh0
                yv[i, pl.ds(0, 16)] = jnp.maximum(y0, 0.0)
                y1 = yv[i, pl.ds(16, 16)] * sc1 + sh1
                yv[i, pl.ds(16, 16)] = jnp.maximum(y1, 0.0)
                return 0

            lax.fori_loop(0, CH2, rowfn, 0)
            for k in range(nsub2):
                pltpu.sync_copy(yv.at[pl.ds(k * KI, KI)],
                                shared.at[tiv.at[half * nsub2 + k]],
                                add=True)
        return 0

    lax.fori_loop(0, nchunks, chunk, 0)
    plsc.subcore_barrier()
    pltpu.sync_copy(shared.at[pl.ds(s * rows_pt, rows_pt)],
                    agg_hbm.at[c].at[pl.ds(s * rows_pt, rows_pt)])


def _pass2(h, ti, ss):
    mesh = plsc.VectorSubcoreMesh(
        core_axis_name="c", subcore_axis_name="s",
        num_cores=NC, num_subcores=NS)
    nsub = CH // KI
    return pl.kernel(
        _pass2_body,
        compiler_params=pltpu.CompilerParams(use_tc_tiling_on_sc=False),
        out_type=jax.ShapeDtypeStruct((NC, NPAD, HH), jnp.float32),
        mesh=mesh,
        scratch_types=[
            pltpu.VMEM_SHARED((NPAD, HH), jnp.float32),
            pltpu.VMEM((nsub, KI), jnp.int32),
            pltpu.VMEM((CH2, HH), jnp.float32),
            pltpu.VMEM((2, HH), jnp.float32),
            pltpu.VMEM((ZCH, HH), jnp.float32),
            pltpu.SemaphoreType.DMA,
        ],
    )(h, ti, ss)


BRD = 5000


def _affine(st_ref, g_ref, be_ref, nrows):
    s = st_ref[0:1, :]
    q = st_ref[1:2, :]
    mu = s / nrows
    var = q / nrows - mu * mu
    sc = g_ref[...] * jax.lax.rsqrt(var + 1e-5)
    sh = be_ref[...] - mu * sc
    return sc, sh


def _acc_stats(i, h, st_ref):
    @pl.when(i == 0)
    def _():
        st_ref[...] = jnp.zeros_like(st_ref)

    st_ref[0:1, :] += jnp.sum(h, axis=0, keepdims=True)
    st_ref[1:2, :] += jnp.sum(h * h, axis=0, keepdims=True)


def _t1_body(agg_ref, x_ref, eps_ref, w_ref, b_ref, h_ref, st_ref):
    i = pl.program_id(0)
    a = jnp.concatenate([agg_ref[0], agg_ref[1]], axis=-1)
    xin = a + (1.0 + eps_ref[0, 0]) * x_ref[...]
    h = jnp.dot(xin, w_ref[...], preferred_element_type=jnp.float32) + b_ref[...]
    h_ref[...] = h
    _acc_stats(i, h, st_ref)


def _t2_body(nrows, st_ref, g_ref, be_ref, h_ref, w_ref, b_ref,
             h2_ref, st2_ref):
    i = pl.program_id(0)
    sc, sh = _affine(st_ref, g_ref, be_ref, nrows)
    x1 = jnp.maximum(h_ref[...] * sc + sh, 0.0)
    h2 = jnp.dot(x1, w_ref[...], preferred_element_type=jnp.float32) + b_ref[...]
    h2_ref[...] = h2
    _acc_stats(i, h2, st2_ref)


def _t3_body(nrows, st_ref, g_ref, be_ref, h_ref, w_ref, p_ref):
    sc, sh = _affine(st_ref, g_ref, be_ref, nrows)
    x2 = jnp.maximum(h_ref[...] * sc + sh, 0.0)
    p_ref[...] = jnp.dot(x2, w_ref[...], preferred_element_type=jnp.float32)


def _t3c_body(nrows, st_ref, g_ref, be_ref, h_ref, w_ref, pu_ref, bco_ref,
              hco_ref, stco_ref):
    i = pl.program_id(0)
    sc, sh = _affine(st_ref, g_ref, be_ref, nrows)
    x2 = jnp.maximum(h_ref[...] * sc + sh, 0.0)
    hco = (pu_ref[...]
           + jnp.dot(x2, w_ref[...], preferred_element_type=jnp.float32)
           + bco_ref[...])
    hco_ref[...] = hco
    _acc_stats(i, hco, stco_ref)


def _t4_body(nrows, st_ref, g_ref, be_ref, h_ref, out_ref):
    sc, sh = _affine(st_ref, g_ref, be_ref, nrows)
    out_ref[...] = jnp.maximum(h_ref[...] * sc + sh, 0.0)


def _row_spec():
    return pl.BlockSpec((BRD, H), lambda i: (i, 0))


def _full_spec(shape):
    return pl.BlockSpec(shape, lambda i: tuple(0 for _ in shape))


def _dense_chain(agg_u, agg_b, x, p):
    n = x.shape[0]
    grid = n // BRD
    nf = float(n)
    hmat = jax.ShapeDtypeStruct((n, H), jnp.float32)
    stat = jax.ShapeDtypeStruct((2, H), jnp.float32)
    stat_spec = pl.BlockSpec((2, H), lambda i: (0, 0))

    def r2(v):
        return v.reshape(1, -1)

    def t1(agg, eps, w, b):
        return pl.pallas_call(
            _t1_body, grid=(grid,),
            in_specs=[pl.BlockSpec((NC, BRD, HH), lambda i: (0, i, 0)),
                      _row_spec(), _full_spec((1, 1)),
                      _full_spec((H, H)), _full_spec((1, H))],
            out_specs=[_row_spec(), stat_spec],
            out_shape=[hmat, stat],
        )(agg, x, eps.reshape(1, 1), w, r2(b))

    def t2(st, g, be, h, w, b):
        return pl.pallas_call(
            functools.partial(_t2_body, nf), grid=(grid,),
            in_specs=[stat_spec, _full_spec((1, H)), _full_spec((1, H)),
                      _row_spec(), _full_spec((H, H)), _full_spec((1, H))],
            out_specs=[_row_spec(), stat_spec],
            out_shape=[hmat, stat],
        )(st, r2(g), r2(be), h, w, r2(b))

    def t3(st, g, be, h, w):
        return pl.pallas_call(
            functools.partial(_t3_body, nf), grid=(grid,),
            in_specs=[stat_spec, _full_spec((1, H)), _full_spec((1, H)),
                      _row_spec(), _full_spec((H, H))],
            out_specs=_row_spec(),
            out_shape=hmat,
        )(st, r2(g), r2(be), h, w)

    def t3c(st, g, be, h, w, pu, bco):
        return pl.pallas_call(
            functools.partial(_t3c_body, nf), grid=(grid,),
            in_specs=[stat_spec, _full_spec((1, H)), _full_spec((1, H)),
                      _row_spec(), _full_spec((H, H)), _row_spec(),
                      _full_spec((1, H))],
            out_specs=[_row_spec(), stat_spec],
            out_shape=[hmat, stat],
        )(st, r2(g), r2(be), h, w, pu, r2(bco))

    def t4(st, g, be, h):
        return pl.pallas_call(
            functools.partial(_t4_body, nf), grid=(grid,),
            in_specs=[stat_spec, _full_spec((1, H)), _full_spec((1, H)),
                      _row_spec()],
            out_specs=_row_spec(),
            out_shape=hmat,
        )(st, r2(g), r2(be), h)

    h1u, s1u = t1(agg_u, p["eps1"], p["uu1_W"], p["uu1_b"])
    h2u, s2u = t2(s1u, p["uu1_g"], p["uu1_be"], h1u, p["uu2_W"], p["uu2_b"])
    pu = t3(s2u, p["uu2_g"], p["uu2_be"], h2u, p["co_W"][:H])

    h1b, s1b = t1(agg_b, p["eps2"], p["ub1_W"], p["ub1_b"])
    h2b, s2b = t2(s1b, p["ub1_g"], p["ub1_be"], h1b, p["ub2_W"], p["ub2_b"])
    hco, sco = t3c(s2b, p["ub2_g"], p["ub2_be"], h2b, p["co_W"][H:], pu,
                   p["co_b"])

    return t4(sco, p["co_g"], p["co_be"], hco)


def kernel(x, up_index, boundary_index, boundary_attr, params):
    p = params
    n = x.shape[0]
    e = up_index.shape[1]

    def r2(v):
        return v.reshape(1, -1)

    xp = jnp.pad(x, ((0, NPAD - n), (0, 0)))
    bap = jnp.pad(boundary_attr, ((0, NPAD - boundary_attr.shape[0]), (0, 0)))
    au, bu, ab, bb = _make_tables(
        xp, bap, p["mu_W"], r2(p["mu_b"]), p["mb_W"], r2(p["mb_b"]), n)

    pad_idx = n + (jnp.arange(EPAD - e, dtype=jnp.int32) % (NPAD - n))

    def prep(idx):
        return jnp.concatenate([idx, pad_idx]).reshape(EPAD // KI, KI)

    ti_u = prep(up_index[0])
    si_u = prep(up_index[1])
    ti_b = prep(boundary_index[1])
    si_b = prep(boundary_index[0])

    h_u, mom_u = _pass1(au, bu, ti_u, si_u)
    h_b, mom_b = _pass1(ab, bb, ti_b, si_b)

    ss_u, ss_b = _mid(mom_u, mom_b,
                      p["mu_g"].reshape(NC, HH), p["mu_be"].reshape(NC, HH),
                      p["mb_g"].reshape(NC, HH), p["mb_be"].reshape(NC, HH),
                      e)

    agg_u = _pass2(h_u, ti_u, ss_u)
    agg_b = _pass2(h_b, ti_b, ss_b)

    return _dense_chain(agg_u, agg_b, x, p)


# speedup vs baseline: 4.7131x; 1.2252x over previous
import functools

import jax
import jax.numpy as jnp
from jax import lax
from jax.experimental import pallas as pl
from jax.experimental.pallas import tpu as pltpu
from jax.experimental.pallas import tpu_sc as plsc

H = 64
HH = 32
NC = 2
NS = 16
KI = 128
CH = 1024
NPAD = 50048
EPAD = 819200
ZCH = 136
CH2 = 512
BR = 3128


def _pre_body(n_real, x_ref, ba_ref, wmu_ref, bmu_ref, wmb_ref, bmb_ref,
              au_ref, bu_ref, ab_ref, bb_ref):
    i = pl.program_id(0)
    rows = i * BR + lax.broadcasted_iota(jnp.int32, (BR, 1), 0)
    valid = (rows < n_real).astype(jnp.float32)
    x = x_ref[...] * valid
    ba = ba_ref[...] * valid
    wmu = wmu_ref[...]
    wmb = wmb_ref[...]
    f32 = jnp.float32
    au_ref[0] = (jnp.dot(x, wmu[:H, :HH], preferred_element_type=f32)
                 + bmu_ref[0, :HH]) * valid
    au_ref[1] = (jnp.dot(x, wmu[:H, HH:], preferred_element_type=f32)
                 + bmu_ref[0, HH:]) * valid
    bu_ref[0] = jnp.dot(x, wmu[H:, :HH], preferred_element_type=f32)
    bu_ref[1] = jnp.dot(x, wmu[H:, HH:], preferred_element_type=f32)
    ab_ref[0] = (jnp.dot(x, wmb[:H, :HH], preferred_element_type=f32)
                 + bmb_ref[0, :HH]) * valid
    ab_ref[1] = (jnp.dot(x, wmb[:H, HH:], preferred_element_type=f32)
                 + bmb_ref[0, HH:]) * valid
    bb_ref[0] = jnp.dot(ba, wmb[H:, :HH], preferred_element_type=f32)
    bb_ref[1] = jnp.dot(ba, wmb[H:, HH:], preferred_element_type=f32)


def _make_tables(x, ba, wmu, bmu, wmb, bmb, n_real):
    grid = NPAD // BR
    tab = jax.ShapeDtypeStruct((NC, NPAD, HH), jnp.float32)
    in_specs = [
        pl.BlockSpec((BR, H), lambda i: (i, 0)),
        pl.BlockSpec((BR, H), lambda i: (i, 0)),
        pl.BlockSpec((2 * H, H), lambda i: (0, 0)),
        pl.BlockSpec((1, H), lambda i: (0, 0)),
        pl.BlockSpec((2 * H, H), lambda i: (0, 0)),
        pl.BlockSpec((1, H), lambda i: (0, 0)),
    ]
    out_spec = pl.BlockSpec((NC, BR, HH), lambda i: (0, i, 0))
    return pl.pallas_call(
        functools.partial(_pre_body, n_real),
        grid=(grid,),
        in_specs=in_specs,
        out_specs=[out_spec] * 4,
        out_shape=[tab] * 4,
    )(x, ba, wmu, bmu, wmb, bmb)


def _pass1_body(a_hbm, b_hbm, ti_hbm, si_hbm, h_hbm, mom_hbm,
                tiv, siv, av, bv, momv, sem):
    c = lax.axis_index("c")
    s = lax.axis_index("s")
    e_pc = EPAD // NS
    nchunks = e_pc // CH
    nsub = CH // KI
    a_tab = a_hbm.at[c]
    b_tab = b_hbm.at[c]
    h_out = h_hbm.at[c]
    row0 = s * (e_pc // KI)

    def chunk(j, carry):
        s0, s1, q0, q1 = carry
        r = row0 + j * nsub
        pltpu.sync_copy(ti_hbm.at[pl.ds(r, nsub)], tiv)
        pltpu.sync_copy(si_hbm.at[pl.ds(r, nsub)], siv)
        descs = []
        for k in range(nsub):
            descs.append(pltpu.async_copy(
                a_tab.at[tiv.at[k]], av.at[pl.ds(k * KI, KI)], sem))
            descs.append(pltpu.async_copy(
                b_tab.at[siv.at[k]], bv.at[pl.ds(k * KI, KI)], sem))
        for d in descs:
            d.wait()

        @plsc.parallel_loop(0, CH, 1, unroll=8, carry=(s0, s1, q0, q1))
        def carry2(i, cr):
            t0, t1, p0, p1 = cr
            h0 = av[i, pl.ds(0, 16)] + bv[i, pl.ds(0, 16)]
            av[i, pl.ds(0, 16)] = h0
            h1 = av[i, pl.ds(16, 16)] + bv[i, pl.ds(16, 16)]
            av[i, pl.ds(16, 16)] = h1
            return (t0 + h0, t1 + h1, p0 + h0 * h0, p1 + h1 * h1)
        pltpu.sync_copy(av, h_out.at[pl.ds(s * e_pc + j * CH, CH)])
        return carry2

    z = jnp.zeros((16,), jnp.float32)
    s0, s1, q0, q1 = lax.fori_loop(0, nchunks, chunk, (z, z, z, z))
    momv[0, pl.ds(0, 16)] = s0
    momv[0, pl.ds(16, 16)] = s1
    momv[1, pl.ds(0, 16)] = q0
    momv[1, pl.ds(16, 16)] = q1
    pltpu.sync_copy(momv, mom_hbm.at[c].at[s])


def _pass1(a_tab, b_tab, ti, si):
    mesh = plsc.VectorSubcoreMesh(
        core_axis_name="c", subcore_axis_name="s",
        num_cores=NC, num_subcores=NS)
    nsub = CH // KI
    return pl.kernel(
        _pass1_body,
        compiler_params=pltpu.CompilerParams(use_tc_tiling_on_sc=False),
        out_type=(
            jax.ShapeDtypeStruct((NC, EPAD, HH), jnp.float32),
            jax.ShapeDtypeStruct((NC, NS, 2, HH), jnp.float32),
        ),
        mesh=mesh,
        scratch_types=[
            pltpu.VMEM((nsub, KI), jnp.int32),
            pltpu.VMEM((nsub, KI), jnp.int32),
            pltpu.VMEM((CH, HH), jnp.float32),
            pltpu.VMEM((CH, HH), jnp.float32),
            pltpu.VMEM((2, HH), jnp.float32),
            pltpu.SemaphoreType.DMA,
        ],
    )(a_tab, b_tab, ti, si)


def _mid_body(mu_ref, mb_ref, gu_ref, beu_ref, gb_ref, beb_ref, e_ref,
              ssu_ref, ssb_ref):
    e = e_ref[0, 0]

    def one(m_ref, g_ref, be_ref, out_ref):
        m = m_ref[...]
        tot = jnp.sum(m, axis=1)
        mu = tot[:, 0, :] / e
        var = tot[:, 1, :] / e - mu * mu
        scale = g_ref[...] * jax.lax.rsqrt(var + 1e-5)
        shift = be_ref[...] - mu * scale
        out_ref[:, 0, :] = scale
        out_ref[:, 1, :] = shift

    one(mu_ref, gu_ref, beu_ref, ssu_ref)
    one(mb_ref, gb_ref, beb_ref, ssb_ref)


def _mid(mom_u, mom_b, gu, beu, gb, beb, e_total):
    e = jnp.full((1, 1), float(e_total), jnp.float32)
    ss = jax.ShapeDtypeStruct((NC, 2, HH), jnp.float32)
    return pl.pallas_call(_mid_body, out_shape=(ss, ss))(
        mom_u, mom_b, gu, beu, gb, beb, e)


def _pass2_body(h_hbm, ti_hbm, ss_hbm, agg_hbm,
                shared, tiv, yv, ssv, zv, sem):
    c = lax.axis_index("c")
    s = lax.axis_index("s")
    e_pc = EPAD // NS
    nchunks = e_pc // CH
    nsub = CH // KI
    rows_pt = NPAD // NS

    pltpu.sync_copy(ss_hbm.at[c], ssv)

    def zrow(i, _):
        zv[i, pl.ds(0, 16)] = jnp.zeros((16,), jnp.float32)
        zv[i, pl.ds(16, 16)] = jnp.zeros((16,), jnp.float32)
        return 0

    lax.fori_loop(0, ZCH, zrow, 0)
    for t in range(rows_pt // ZCH):
        pltpu.sync_copy(zv, shared.at[pl.ds(s * rows_pt + t * ZCH, ZCH)])
    plsc.subcore_barrier()

    sc0 = ssv[0, pl.ds(0, 16)]
    sc1 = ssv[0, pl.ds(16, 16)]
    sh0 = ssv[1, pl.ds(0, 16)]
    sh1 = ssv[1, pl.ds(16, 16)]
    h_in = h_hbm.at[c]
    row0 = s * (e_pc // KI)
    nsub2 = CH2 // KI

    def chunk(j, _):
        r = row0 + j * nsub
        pltpu.sync_copy(ti_hbm.at[pl.ds(r, nsub)], tiv)
        for half in range(CH // CH2):
            pltpu.sync_copy(
                h_in.at[pl.ds(s * e_pc + j * CH + half * CH2, CH2)], yv)

            @plsc.parallel_loop(0, CH2, 1, unroll=8)
            def _rows(i):
                y0 = yv[i, pl.ds(0, 16)] * sc0 + sh0
                yv[i, pl.ds(0, 16)] = jnp.maximum(y0, 0.0)
                y1 = yv[i, pl.ds(16, 16)] * sc1 + sh1
                yv[i, pl.ds(16, 16)] = jnp.maximum(y1, 0.0)
            for k in range(nsub2):
                pltpu.sync_copy(yv.at[pl.ds(k * KI, KI)],
                                shared.at[tiv.at[half * nsub2 + k]],
                                add=True)
        return 0

    lax.fori_loop(0, nchunks, chunk, 0)
    plsc.subcore_barrier()
    pltpu.sync_copy(shared.at[pl.ds(s * rows_pt, rows_pt)],
                    agg_hbm.at[c].at[pl.ds(s * rows_pt, rows_pt)])


def _pass2(h, ti, ss):
    mesh = plsc.VectorSubcoreMesh(
        core_axis_name="c", subcore_axis_name="s",
        num_cores=NC, num_subcores=NS)
    nsub = CH // KI
    return pl.kernel(
        _pass2_body,
        compiler_params=pltpu.CompilerParams(use_tc_tiling_on_sc=False),
        out_type=jax.ShapeDtypeStruct((NC, NPAD, HH), jnp.float32),
        mesh=mesh,
        scratch_types=[
            pltpu.VMEM_SHARED((NPAD, HH), jnp.float32),
            pltpu.VMEM((nsub, KI), jnp.int32),
            pltpu.VMEM((CH2, HH), jnp.float32),
            pltpu.VMEM((2, HH), jnp.float32),
            pltpu.VMEM((ZCH, HH), jnp.float32),
            pltpu.SemaphoreType.DMA,
        ],
    )(h, ti, ss)


BRD = 5000


def _affine(st_ref, g_ref, be_ref, nrows):
    s = st_ref[0:1, :]
    q = st_ref[1:2, :]
    mu = s / nrows
    var = q / nrows - mu * mu
    sc = g_ref[...] * jax.lax.rsqrt(var + 1e-5)
    sh = be_ref[...] - mu * sc
    return sc, sh


def _acc_stats(i, h, st_ref):
    @pl.when(i == 0)
    def _():
        st_ref[...] = jnp.zeros_like(st_ref)

    st_ref[0:1, :] += jnp.sum(h, axis=0, keepdims=True)
    st_ref[1:2, :] += jnp.sum(h * h, axis=0, keepdims=True)


def _t1_body(agg_ref, x_ref, eps_ref, w_ref, b_ref, h_ref, st_ref):
    i = pl.program_id(0)
    a = jnp.concatenate([agg_ref[0], agg_ref[1]], axis=-1)
    xin = a + (1.0 + eps_ref[0, 0]) * x_ref[...]
    h = jnp.dot(xin, w_ref[...], preferred_element_type=jnp.float32) + b_ref[...]
    h_ref[...] = h
    _acc_stats(i, h, st_ref)


def _t2_body(nrows, st_ref, g_ref, be_ref, h_ref, w_ref, b_ref,
             h2_ref, st2_ref):
    i = pl.program_id(0)
    sc, sh = _affine(st_ref, g_ref, be_ref, nrows)
    x1 = jnp.maximum(h_ref[...] * sc + sh, 0.0)
    h2 = jnp.dot(x1, w_ref[...], preferred_element_type=jnp.float32) + b_ref[...]
    h2_ref[...] = h2
    _acc_stats(i, h2, st2_ref)


def _t3_body(nrows, st_ref, g_ref, be_ref, h_ref, w_ref, p_ref):
    sc, sh = _affine(st_ref, g_ref, be_ref, nrows)
    x2 = jnp.maximum(h_ref[...] * sc + sh, 0.0)
    p_ref[...] = jnp.dot(x2, w_ref[...], preferred_element_type=jnp.float32)


def _t3c_body(nrows, st_ref, g_ref, be_ref, h_ref, w_ref, pu_ref, bco_ref,
              hco_ref, stco_ref):
    i = pl.program_id(0)
    sc, sh = _affine(st_ref, g_ref, be_ref, nrows)
    x2 = jnp.maximum(h_ref[...] * sc + sh, 0.0)
    hco = (pu_ref[...]
           + jnp.dot(x2, w_ref[...], preferred_element_type=jnp.float32)
           + bco_ref[...])
    hco_ref[...] = hco
    _acc_stats(i, hco, stco_ref)


def _t4_body(nrows, st_ref, g_ref, be_ref, h_ref, out_ref):
    sc, sh = _affine(st_ref, g_ref, be_ref, nrows)
    out_ref[...] = jnp.maximum(h_ref[...] * sc + sh, 0.0)


def _row_spec():
    return pl.BlockSpec((BRD, H), lambda i: (i, 0))


def _full_spec(shape):
    return pl.BlockSpec(shape, lambda i: tuple(0 for _ in shape))


def _dense_chain(agg_u, agg_b, x, p):
    n = x.shape[0]
    grid = n // BRD
    nf = float(n)
    hmat = jax.ShapeDtypeStruct((n, H), jnp.float32)
    stat = jax.ShapeDtypeStruct((2, H), jnp.float32)
    stat_spec = pl.BlockSpec((2, H), lambda i: (0, 0))

    def r2(v):
        return v.reshape(1, -1)

    def t1(agg, eps, w, b):
        return pl.pallas_call(
            _t1_body, grid=(grid,),
            in_specs=[pl.BlockSpec((NC, BRD, HH), lambda i: (0, i, 0)),
                      _row_spec(), _full_spec((1, 1)),
                      _full_spec((H, H)), _full_spec((1, H))],
            out_specs=[_row_spec(), stat_spec],
            out_shape=[hmat, stat],
        )(agg, x, eps.reshape(1, 1), w, r2(b))

    def t2(st, g, be, h, w, b):
        return pl.pallas_call(
            functools.partial(_t2_body, nf), grid=(grid,),
            in_specs=[stat_spec, _full_spec((1, H)), _full_spec((1, H)),
                      _row_spec(), _full_spec((H, H)), _full_spec((1, H))],
            out_specs=[_row_spec(), stat_spec],
            out_shape=[hmat, stat],
        )(st, r2(g), r2(be), h, w, r2(b))

    def t3(st, g, be, h, w):
        return pl.pallas_call(
            functools.partial(_t3_body, nf), grid=(grid,),
            in_specs=[stat_spec, _full_spec((1, H)), _full_spec((1, H)),
                      _row_spec(), _full_spec((H, H))],
            out_specs=_row_spec(),
            out_shape=hmat,
        )(st, r2(g), r2(be), h, w)

    def t3c(st, g, be, h, w, pu, bco):
        return pl.pallas_call(
            functools.partial(_t3c_body, nf), grid=(grid,),
            in_specs=[stat_spec, _full_spec((1, H)), _full_spec((1, H)),
                      _row_spec(), _full_spec((H, H)), _row_spec(),
                      _full_spec((1, H))],
            out_specs=[_row_spec(), stat_spec],
            out_shape=[hmat, stat],
        )(st, r2(g), r2(be), h, w, pu, r2(bco))

    def t4(st, g, be, h):
        return pl.pallas_call(
            functools.partial(_t4_body, nf), grid=(grid,),
            in_specs=[stat_spec, _full_spec((1, H)), _full_spec((1, H)),
                      _row_spec()],
            out_specs=_row_spec(),
            out_shape=hmat,
        )(st, r2(g), r2(be), h)

    h1u, s1u = t1(agg_u, p["eps1"], p["uu1_W"], p["uu1_b"])
    h2u, s2u = t2(s1u, p["uu1_g"], p["uu1_be"], h1u, p["uu2_W"], p["uu2_b"])
    pu = t3(s2u, p["uu2_g"], p["uu2_be"], h2u, p["co_W"][:H])

    h1b, s1b = t1(agg_b, p["eps2"], p["ub1_W"], p["ub1_b"])
    h2b, s2b = t2(s1b, p["ub1_g"], p["ub1_be"], h1b, p["ub2_W"], p["ub2_b"])
    hco, sco = t3c(s2b, p["ub2_g"], p["ub2_be"], h2b, p["co_W"][H:], pu,
                   p["co_b"])

    return t4(sco, p["co_g"], p["co_be"], hco)


def kernel(x, up_index, boundary_index, boundary_attr, params):
    p = params
    n = x.shape[0]
    e = up_index.shape[1]

    def r2(v):
        return v.reshape(1, -1)

    xp = jnp.pad(x, ((0, NPAD - n), (0, 0)))
    bap = jnp.pad(boundary_attr, ((0, NPAD - boundary_attr.shape[0]), (0, 0)))
    au, bu, ab, bb = _make_tables(
        xp, bap, p["mu_W"], r2(p["mu_b"]), p["mb_W"], r2(p["mb_b"]), n)

    pad_idx = n + (jnp.arange(EPAD - e, dtype=jnp.int32) % (NPAD - n))

    def prep(idx):
        return jnp.concatenate([idx, pad_idx]).reshape(EPAD // KI, KI)

    ti_u = prep(up_index[0])
    si_u = prep(up_index[1])
    ti_b = prep(boundary_index[1])
    si_b = prep(boundary_index[0])

    h_u, mom_u = _pass1(au, bu, ti_u, si_u)
    h_b, mom_b = _pass1(ab, bb, ti_b, si_b)

    ss_u, ss_b = _mid(mom_u, mom_b,
                      p["mu_g"].reshape(NC, HH), p["mu_be"].reshape(NC, HH),
                      p["mb_g"].reshape(NC, HH), p["mb_be"].reshape(NC, HH),
                      e)

    agg_u = _pass2(h_u, ti_u, ss_u)
    agg_b = _pass2(h_b, ti_b, ss_b)

    return _dense_chain(agg_u, agg_b, x, p)


# speedup vs baseline: 5.6346x; 1.1955x over previous
import functools

import jax
import jax.numpy as jnp
from jax import lax
from jax.experimental import pallas as pl
from jax.experimental.pallas import tpu as pltpu
from jax.experimental.pallas import tpu_sc as plsc

H = 64
HH = 32
NC = 2
NS = 16
KI = 128
CH = 1024
NPAD = 50048
EPAD = 819200
ZCH = 136
CH2 = 512
BR = 3128


def _pre_body(n_real, x_ref, ba_ref, wmu_ref, bmu_ref, wmb_ref, bmb_ref,
              au_ref, bu_ref, ab_ref, bb_ref):
    i = pl.program_id(0)
    rows = i * BR + lax.broadcasted_iota(jnp.int32, (BR, 1), 0)
    valid = (rows < n_real).astype(jnp.float32)
    x = x_ref[...] * valid
    ba = ba_ref[...] * valid
    wmu = wmu_ref[...]
    wmb = wmb_ref[...]
    f32 = jnp.float32
    au_ref[0] = (jnp.dot(x, wmu[:H, :HH], preferred_element_type=f32)
                 + bmu_ref[0, :HH]) * valid
    au_ref[1] = (jnp.dot(x, wmu[:H, HH:], preferred_element_type=f32)
                 + bmu_ref[0, HH:]) * valid
    bu_ref[0] = jnp.dot(x, wmu[H:, :HH], preferred_element_type=f32)
    bu_ref[1] = jnp.dot(x, wmu[H:, HH:], preferred_element_type=f32)
    ab_ref[0] = (jnp.dot(x, wmb[:H, :HH], preferred_element_type=f32)
                 + bmb_ref[0, :HH]) * valid
    ab_ref[1] = (jnp.dot(x, wmb[:H, HH:], preferred_element_type=f32)
                 + bmb_ref[0, HH:]) * valid
    bb_ref[0] = jnp.dot(ba, wmb[H:, :HH], preferred_element_type=f32)
    bb_ref[1] = jnp.dot(ba, wmb[H:, HH:], preferred_element_type=f32)


def _make_tables(x, ba, wmu, bmu, wmb, bmb, n_real):
    grid = NPAD // BR
    tab = jax.ShapeDtypeStruct((NC, NPAD, HH), jnp.float32)
    in_specs = [
        pl.BlockSpec((BR, H), lambda i: (i, 0)),
        pl.BlockSpec((BR, H), lambda i: (i, 0)),
        pl.BlockSpec((2 * H, H), lambda i: (0, 0)),
        pl.BlockSpec((1, H), lambda i: (0, 0)),
        pl.BlockSpec((2 * H, H), lambda i: (0, 0)),
        pl.BlockSpec((1, H), lambda i: (0, 0)),
    ]
    out_spec = pl.BlockSpec((NC, BR, HH), lambda i: (0, i, 0))
    return pl.pallas_call(
        functools.partial(_pre_body, n_real),
        grid=(grid,),
        in_specs=in_specs,
        out_specs=[out_spec] * 4,
        out_shape=[tab] * 4,
    )(x, ba, wmu, bmu, wmb, bmb)


C1 = 512
NS1 = C1 // KI


def _p1_issue(jj, p, a_tab, b_tab, ti_hbm, si_hbm, tiv, siv, av, bv,
              sems, row0):
    r = row0 + jj * NS1
    pltpu.sync_copy(ti_hbm.at[pl.ds(r, NS1)], tiv.at[pl.ds(p * NS1, NS1)])
    pltpu.sync_copy(si_hbm.at[pl.ds(r, NS1)], siv.at[pl.ds(p * NS1, NS1)])
    for k in range(NS1):
        pltpu.async_copy(a_tab.at[tiv.at[p * NS1 + k]],
                         av.at[pl.ds(p * C1 + k * KI, KI)], sems[p])
        pltpu.async_copy(b_tab.at[siv.at[p * NS1 + k]],
                         bv.at[pl.ds(p * C1 + k * KI, KI)], sems[p])


def _p1_wait_gather(p, h_out, av, bv, sems):
    pltpu.make_async_copy(h_out.at[pl.ds(0, C1)],
                          av.at[pl.ds(p * C1, C1)], sems[p]).wait()
    pltpu.make_async_copy(h_out.at[pl.ds(0, C1)],
                          bv.at[pl.ds(p * C1, C1)], sems[p]).wait()


def _p1_wait_wb(p, h_out, av, sems):
    pltpu.make_async_copy(h_out.at[pl.ds(0, C1)],
                          av.at[pl.ds(p * C1, C1)], sems[2 + p]).wait()


def _pass1_body(a_hbm, b_hbm, ti_hbm, si_hbm, h_hbm, mom_hbm,
                tiv, siv, av, bv, momv, sg0, sg1, sw0, sw1):
    c = lax.axis_index("c")
    s = lax.axis_index("s")
    e_pc = EPAD // NS
    nchunks = e_pc // C1
    a_tab = a_hbm.at[c]
    b_tab = b_hbm.at[c]
    h_out = h_hbm.at[c]
    row0 = s * (e_pc // KI)
    sems = (sg0, sg1, sw0, sw1)

    _p1_issue(0, 0, a_tab, b_tab, ti_hbm, si_hbm, tiv, siv, av, bv,
              sems, row0)

    def step(jj, p, carry):
        np_ = 1 - p

        @pl.when(jj + 1 < nchunks)
        def _():
            @pl.when(jj >= 1)
            def _():
                _p1_wait_wb(np_, h_out, av, sems)
            _p1_issue(jj + 1, np_, a_tab, b_tab, ti_hbm, si_hbm,
                      tiv, siv, av, bv, sems, row0)

        _p1_wait_gather(p, h_out, av, bv, sems)

        @plsc.parallel_loop(0, C1, 1, unroll=8, carry=carry)
        def carry2(i, cr):
            t0, t1, p0, p1 = cr
            h0 = av[p * C1 + i, pl.ds(0, 16)] + bv[p * C1 + i, pl.ds(0, 16)]
            av[p * C1 + i, pl.ds(0, 16)] = h0
            h1 = (av[p * C1 + i, pl.ds(16, 16)]
                  + bv[p * C1 + i, pl.ds(16, 16)])
            av[p * C1 + i, pl.ds(16, 16)] = h1
            return (t0 + h0, t1 + h1, p0 + h0 * h0, p1 + h1 * h1)

        pltpu.async_copy(av.at[pl.ds(p * C1, C1)],
                         h_out.at[pl.ds(s * e_pc + jj * C1, C1)],
                         sems[2 + p])
        return carry2

    def pair(q, carry):
        carry = step(2 * q, 0, carry)
        carry = step(2 * q + 1, 1, carry)
        return carry

    z = jnp.zeros((16,), jnp.float32)
    s0, s1, q0, q1 = lax.fori_loop(0, nchunks // 2, pair, (z, z, z, z))
    _p1_wait_wb(0, h_out, av, sems)
    _p1_wait_wb(1, h_out, av, sems)
    momv[0, pl.ds(0, 16)] = s0
    momv[0, pl.ds(16, 16)] = s1
    momv[1, pl.ds(0, 16)] = q0
    momv[1, pl.ds(16, 16)] = q1
    pltpu.sync_copy(momv, mom_hbm.at[c].at[s])


def _pass1(a_tab, b_tab, ti, si):
    mesh = plsc.VectorSubcoreMesh(
        core_axis_name="c", subcore_axis_name="s",
        num_cores=NC, num_subcores=NS)
    return pl.kernel(
        _pass1_body,
        compiler_params=pltpu.CompilerParams(use_tc_tiling_on_sc=False),
        out_type=(
            jax.ShapeDtypeStruct((NC, EPAD, HH), jnp.float32),
            jax.ShapeDtypeStruct((NC, NS, 2, HH), jnp.float32),
        ),
        mesh=mesh,
        scratch_types=[
            pltpu.VMEM((2 * NS1, KI), jnp.int32),
            pltpu.VMEM((2 * NS1, KI), jnp.int32),
            pltpu.VMEM((2 * C1, HH), jnp.float32),
            pltpu.VMEM((2 * C1, HH), jnp.float32),
            pltpu.VMEM((2, HH), jnp.float32),
            pltpu.SemaphoreType.DMA,
            pltpu.SemaphoreType.DMA,
            pltpu.SemaphoreType.DMA,
            pltpu.SemaphoreType.DMA,
        ],
    )(a_tab, b_tab, ti, si)


def _mid_body(m_ref, g_ref, be_ref, e_ref, out_ref):
    e = e_ref[0, 0]
    m = m_ref[...]
    tot = jnp.sum(m, axis=1)
    mu = tot[:, 0, :] / e
    var = tot[:, 1, :] / e - mu * mu
    scale = g_ref[...] * jax.lax.rsqrt(var + 1e-5)
    shift = be_ref[...] - mu * scale
    out_ref[:, 0, :] = scale
    out_ref[:, 1, :] = shift


def _mid(mom, g, be, e_total):
    e = jnp.full((1, 1), float(e_total), jnp.float32)
    ss = jax.ShapeDtypeStruct((NC, 2, HH), jnp.float32)
    return pl.pallas_call(_mid_body, out_shape=ss)(mom, g, be, e)


C2 = 256
NS2 = C2 // KI


def _p2_issue(jj, p, h_in, ti_hbm, tiv, yv, sems, row0, base):
    pltpu.sync_copy(ti_hbm.at[pl.ds(row0 + jj * NS2, NS2)],
                    tiv.at[pl.ds(p * NS2, NS2)])
    pltpu.async_copy(h_in.at[pl.ds(base + jj * C2, C2)],
                     yv.at[pl.ds(p * C2, C2)], sems[p])


def _p2_wait_h(p, h_in, yv, sems):
    pltpu.make_async_copy(h_in.at[pl.ds(0, C2)],
                          yv.at[pl.ds(p * C2, C2)], sems[p]).wait()


def _p2_wait_scat(p, shared, yv, sems):
    pltpu.make_async_copy(yv.at[pl.ds(0, C2)],
                          shared.at[pl.ds(0, C2)], sems[2 + p]).wait()


def _pass2_body(h_hbm, ti_hbm, ss_hbm, agg_hbm,
                shared, tiv, yv, ssv, sg0, sg1, ss0, ss1):
    c = lax.axis_index("c")
    s = lax.axis_index("s")
    e_pc = EPAD // NS
    nchunks = e_pc // C2
    rows_pt = NPAD // NS
    sems = (sg0, sg1, ss0, ss1)

    pltpu.sync_copy(ss_hbm.at[c], ssv)

    @plsc.parallel_loop(0, 2 * C2, 1, unroll=8)
    def _z(i):
        yv[i, pl.ds(0, 16)] = jnp.zeros((16,), jnp.float32)
        yv[i, pl.ds(16, 16)] = jnp.zeros((16,), jnp.float32)

    stripe0 = s * rows_pt
    for t in range(rows_pt // (2 * C2)):
        pltpu.sync_copy(yv, shared.at[pl.ds(stripe0 + t * 2 * C2, 2 * C2)])
    rem0 = (rows_pt // (2 * C2)) * 2 * C2
    pltpu.sync_copy(yv.at[pl.ds(0, rows_pt - rem0)],
                    shared.at[pl.ds(stripe0 + rem0, rows_pt - rem0)])
    plsc.subcore_barrier()

    sc0 = ssv[0, pl.ds(0, 16)]
    sc1 = ssv[0, pl.ds(16, 16)]
    sh0 = ssv[1, pl.ds(0, 16)]
    sh1 = ssv[1, pl.ds(16, 16)]
    h_in = h_hbm.at[c]
    row0 = s * (e_pc // KI)
    base = s * e_pc

    _p2_issue(0, 0, h_in, ti_hbm, tiv, yv, sems, row0, base)

    def step(jj, p):
        np_ = 1 - p

        @pl.when(jj >= 1)
        def _():
            _p2_wait_scat(np_, shared, yv, sems)

        @pl.when(jj + 1 < nchunks)
        def _():
            _p2_issue(jj + 1, np_, h_in, ti_hbm, tiv, yv, sems, row0, base)

        _p2_wait_h(p, h_in, yv, sems)

        @plsc.parallel_loop(0, C2, 1, unroll=8)
        def _rows(i):
            y0 = yv[p * C2 + i, pl.ds(0, 16)] * sc0 + sh0
            yv[p * C2 + i, pl.ds(0, 16)] = jnp.maximum(y0, 0.0)
            y1 = yv[p * C2 + i, pl.ds(16, 16)] * sc1 + sh1
            yv[p * C2 + i, pl.ds(16, 16)] = jnp.maximum(y1, 0.0)

        for k in range(NS2):
            pltpu.async_copy(yv.at[pl.ds(p * C2 + k * KI, KI)],
                             shared.at[tiv.at[p * NS2 + k]],
                             sems[2 + p], add=True)

    def pair(q, _):
        step(2 * q, 0)
        step(2 * q + 1, 1)
        return 0

    lax.fori_loop(0, nchunks // 2, pair, 0)
    _p2_wait_scat(1, shared, yv, sems)
    plsc.subcore_barrier()
    pltpu.sync_copy(shared.at[pl.ds(s * rows_pt, rows_pt)],
                    agg_hbm.at[c].at[pl.ds(s * rows_pt, rows_pt)])


def _pass2(h, ti, ss):
    mesh = plsc.VectorSubcoreMesh(
        core_axis_name="c", subcore_axis_name="s",
        num_cores=NC, num_subcores=NS)
    return pl.kernel(
        _pass2_body,
        compiler_params=pltpu.CompilerParams(use_tc_tiling_on_sc=False),
        out_type=jax.ShapeDtypeStruct((NC, NPAD, HH), jnp.float32),
        mesh=mesh,
        scratch_types=[
            pltpu.VMEM_SHARED((NPAD, HH), jnp.float32),
            pltpu.VMEM((2 * NS2, KI), jnp.int32),
            pltpu.VMEM((2 * C2, HH), jnp.float32),
            pltpu.VMEM((2, HH), jnp.float32),
            pltpu.SemaphoreType.DMA,
            pltpu.SemaphoreType.DMA,
            pltpu.SemaphoreType.DMA,
            pltpu.SemaphoreType.DMA,
        ],
    )(h, ti, ss)


BRD = 5000


def _affine(st_ref, g_ref, be_ref, nrows):
    s = st_ref[0:1, :]
    q = st_ref[1:2, :]
    mu = s / nrows
    var = q / nrows - mu * mu
    sc = g_ref[...] * jax.lax.rsqrt(var + 1e-5)
    sh = be_ref[...] - mu * sc
    return sc, sh


def _acc_stats(i, h, st_ref):
    @pl.when(i == 0)
    def _():
        st_ref[...] = jnp.zeros_like(st_ref)

    st_ref[0:1, :] += jnp.sum(h, axis=0, keepdims=True)
    st_ref[1:2, :] += jnp.sum(h * h, axis=0, keepdims=True)


def _t1_body(agg_ref, x_ref, eps_ref, w_ref, b_ref, h_ref, st_ref):
    i = pl.program_id(0)
    a = jnp.concatenate([agg_ref[0], agg_ref[1]], axis=-1)
    xin = a + (1.0 + eps_ref[0, 0]) * x_ref[...]
    h = jnp.dot(xin, w_ref[...], preferred_element_type=jnp.float32) + b_ref[...]
    h_ref[...] = h
    _acc_stats(i, h, st_ref)


def _t2_body(nrows, st_ref, g_ref, be_ref, h_ref, w_ref, b_ref,
             h2_ref, st2_ref):
    i = pl.program_id(0)
    sc, sh = _affine(st_ref, g_ref, be_ref, nrows)
    x1 = jnp.maximum(h_ref[...] * sc + sh, 0.0)
    h2 = jnp.dot(x1, w_ref[...], preferred_element_type=jnp.float32) + b_ref[...]
    h2_ref[...] = h2
    _acc_stats(i, h2, st2_ref)


def _t3_body(nrows, st_ref, g_ref, be_ref, h_ref, w_ref, p_ref):
    sc, sh = _affine(st_ref, g_ref, be_ref, nrows)
    x2 = jnp.maximum(h_ref[...] * sc + sh, 0.0)
    p_ref[...] = jnp.dot(x2, w_ref[...], preferred_element_type=jnp.float32)


def _t3c_body(nrows, st_ref, g_ref, be_ref, h_ref, w_ref, pu_ref, bco_ref,
              hco_ref, stco_ref):
    i = pl.program_id(0)
    sc, sh = _affine(st_ref, g_ref, be_ref, nrows)
    x2 = jnp.maximum(h_ref[...] * sc + sh, 0.0)
    hco = (pu_ref[...]
           + jnp.dot(x2, w_ref[...], preferred_element_type=jnp.float32)
           + bco_ref[...])
    hco_ref[...] = hco
    _acc_stats(i, hco, stco_ref)


def _t4_body(nrows, st_ref, g_ref, be_ref, h_ref, out_ref):
    sc, sh = _affine(st_ref, g_ref, be_ref, nrows)
    out_ref[...] = jnp.maximum(h_ref[...] * sc + sh, 0.0)


def _row_spec():
    return pl.BlockSpec((BRD, H), lambda i: (i, 0))


def _full_spec(shape):
    return pl.BlockSpec(shape, lambda i: tuple(0 for _ in shape))


def _dense_chain(agg_u, agg_b, x, p):
    n = x.shape[0]
    grid = n // BRD
    nf = float(n)
    hmat = jax.ShapeDtypeStruct((n, H), jnp.float32)
    stat = jax.ShapeDtypeStruct((2, H), jnp.float32)
    stat_spec = pl.BlockSpec((2, H), lambda i: (0, 0))

    def r2(v):
        return v.reshape(1, -1)

    def t1(agg, eps, w, b):
        return pl.pallas_call(
            _t1_body, grid=(grid,),
            in_specs=[pl.BlockSpec((NC, BRD, HH), lambda i: (0, i, 0)),
                      _row_spec(), _full_spec((1, 1)),
                      _full_spec((H, H)), _full_spec((1, H))],
            out_specs=[_row_spec(), stat_spec],
            out_shape=[hmat, stat],
        )(agg, x, eps.reshape(1, 1), w, r2(b))

    def t2(st, g, be, h, w, b):
        return pl.pallas_call(
            functools.partial(_t2_body, nf), grid=(grid,),
            in_specs=[stat_spec, _full_spec((1, H)), _full_spec((1, H)),
                      _row_spec(), _full_spec((H, H)), _full_spec((1, H))],
            out_specs=[_row_spec(), stat_spec],
            out_shape=[hmat, stat],
        )(st, r2(g), r2(be), h, w, r2(b))

    def t3(st, g, be, h, w):
        return pl.pallas_call(
            functools.partial(_t3_body, nf), grid=(grid,),
            in_specs=[stat_spec, _full_spec((1, H)), _full_spec((1, H)),
                      _row_spec(), _full_spec((H, H))],
            out_specs=_row_spec(),
            out_shape=hmat,
        )(st, r2(g), r2(be), h, w)

    def t3c(st, g, be, h, w, pu, bco):
        return pl.pallas_call(
            functools.partial(_t3c_body, nf), grid=(grid,),
            in_specs=[stat_spec, _full_spec((1, H)), _full_spec((1, H)),
                      _row_spec(), _full_spec((H, H)), _row_spec(),
                      _full_spec((1, H))],
            out_specs=[_row_spec(), stat_spec],
            out_shape=[hmat, stat],
        )(st, r2(g), r2(be), h, w, pu, r2(bco))

    def t4(st, g, be, h):
        return pl.pallas_call(
            functools.partial(_t4_body, nf), grid=(grid,),
            in_specs=[stat_spec, _full_spec((1, H)), _full_spec((1, H)),
                      _row_spec()],
            out_specs=_row_spec(),
            out_shape=hmat,
        )(st, r2(g), r2(be), h)

    h1u, s1u = t1(agg_u, p["eps1"], p["uu1_W"], p["uu1_b"])
    h2u, s2u = t2(s1u, p["uu1_g"], p["uu1_be"], h1u, p["uu2_W"], p["uu2_b"])
    pu = t3(s2u, p["uu2_g"], p["uu2_be"], h2u, p["co_W"][:H])

    h1b, s1b = t1(agg_b, p["eps2"], p["ub1_W"], p["ub1_b"])
    h2b, s2b = t2(s1b, p["ub1_g"], p["ub1_be"], h1b, p["ub2_W"], p["ub2_b"])
    hco, sco = t3c(s2b, p["ub2_g"], p["ub2_be"], h2b, p["co_W"][H:], pu,
                   p["co_b"])

    return t4(sco, p["co_g"], p["co_be"], hco)


def kernel(x, up_index, boundary_index, boundary_attr, params):
    p = params
    n = x.shape[0]
    e = up_index.shape[1]

    def r2(v):
        return v.reshape(1, -1)

    xp = jnp.pad(x, ((0, NPAD - n), (0, 0)))
    bap = jnp.pad(boundary_attr, ((0, NPAD - boundary_attr.shape[0]), (0, 0)))
    au, bu, ab, bb = _make_tables(
        xp, bap, p["mu_W"], r2(p["mu_b"]), p["mb_W"], r2(p["mb_b"]), n)

    pad_idx = n + (jnp.arange(EPAD - e, dtype=jnp.int32) % (NPAD - n))

    def prep(idx):
        return jnp.concatenate([idx, pad_idx]).reshape(EPAD // KI, KI)

    ti_u = prep(up_index[0])
    si_u = prep(up_index[1])
    ti_b = prep(boundary_index[1])
    si_b = prep(boundary_index[0])

    h_u, mom_u = _pass1(au, bu, ti_u, si_u)
    ss_u = _mid(mom_u, p["mu_g"].reshape(NC, HH),
                p["mu_be"].reshape(NC, HH), e)
    agg_u = _pass2(h_u, ti_u, ss_u)

    h_b, mom_b = _pass1(ab, bb, ti_b, si_b)
    ss_b = _mid(mom_b, p["mb_g"].reshape(NC, HH),
                p["mb_be"].reshape(NC, HH), e)
    agg_b = _pass2(h_b, ti_b, ss_b)

    return _dense_chain(agg_u, agg_b, x, p)


# speedup vs baseline: 5.6370x; 1.0004x over previous
import functools

import jax
import jax.numpy as jnp
from jax import lax
from jax.experimental import pallas as pl
from jax.experimental.pallas import tpu as pltpu
from jax.experimental.pallas import tpu_sc as plsc

H = 64
HH = 32
NC = 2
NS = 16
KI1 = 512
KI2 = 256
CH = 1024
NPAD = 50048
EPAD = 819200
ZCH = 136
CH2 = 512
BR = 3128


def _pre_body(n_real, x_ref, ba_ref, wmu_ref, bmu_ref, wmb_ref, bmb_ref,
              au_ref, bu_ref, ab_ref, bb_ref):
    i = pl.program_id(0)
    rows = i * BR + lax.broadcasted_iota(jnp.int32, (BR, 1), 0)
    valid = (rows < n_real).astype(jnp.float32)
    x = x_ref[...] * valid
    ba = ba_ref[...] * valid
    wmu = wmu_ref[...]
    wmb = wmb_ref[...]
    f32 = jnp.float32
    au_ref[0] = (jnp.dot(x, wmu[:H, :HH], preferred_element_type=f32)
                 + bmu_ref[0, :HH]) * valid
    au_ref[1] = (jnp.dot(x, wmu[:H, HH:], preferred_element_type=f32)
                 + bmu_ref[0, HH:]) * valid
    bu_ref[0] = jnp.dot(x, wmu[H:, :HH], preferred_element_type=f32)
    bu_ref[1] = jnp.dot(x, wmu[H:, HH:], preferred_element_type=f32)
    ab_ref[0] = (jnp.dot(x, wmb[:H, :HH], preferred_element_type=f32)
                 + bmb_ref[0, :HH]) * valid
    ab_ref[1] = (jnp.dot(x, wmb[:H, HH:], preferred_element_type=f32)
                 + bmb_ref[0, HH:]) * valid
    bb_ref[0] = jnp.dot(ba, wmb[H:, :HH], preferred_element_type=f32)
    bb_ref[1] = jnp.dot(ba, wmb[H:, HH:], preferred_element_type=f32)


def _make_tables(x, ba, wmu, bmu, wmb, bmb, n_real):
    grid = NPAD // BR
    tab = jax.ShapeDtypeStruct((NC, NPAD, HH), jnp.float32)
    in_specs = [
        pl.BlockSpec((BR, H), lambda i: (i, 0)),
        pl.BlockSpec((BR, H), lambda i: (i, 0)),
        pl.BlockSpec((2 * H, H), lambda i: (0, 0)),
        pl.BlockSpec((1, H), lambda i: (0, 0)),
        pl.BlockSpec((2 * H, H), lambda i: (0, 0)),
        pl.BlockSpec((1, H), lambda i: (0, 0)),
    ]
    out_spec = pl.BlockSpec((NC, BR, HH), lambda i: (0, i, 0))
    return pl.pallas_call(
        functools.partial(_pre_body, n_real),
        grid=(grid,),
        in_specs=in_specs,
        out_specs=[out_spec] * 4,
        out_shape=[tab] * 4,
    )(x, ba, wmu, bmu, wmb, bmb)


C1 = 512
NS1 = C1 // KI1


def _p1_issue(jj, p, a_tab, b_tab, ti_hbm, si_hbm, tiv, siv, av, bv,
              sems, row0):
    r = row0 + jj * NS1
    pltpu.sync_copy(ti_hbm.at[pl.ds(r, NS1)], tiv.at[pl.ds(p * NS1, NS1)])
    pltpu.sync_copy(si_hbm.at[pl.ds(r, NS1)], siv.at[pl.ds(p * NS1, NS1)])
    for k in range(NS1):
        pltpu.async_copy(a_tab.at[tiv.at[p * NS1 + k]],
                         av.at[pl.ds(p * C1 + k * KI1, KI1)], sems[p])
        pltpu.async_copy(b_tab.at[siv.at[p * NS1 + k]],
                         bv.at[pl.ds(p * C1 + k * KI1, KI1)], sems[p])


def _p1_wait_gather(p, h_out, av, bv, sems):
    pltpu.make_async_copy(h_out.at[pl.ds(0, C1)],
                          av.at[pl.ds(p * C1, C1)], sems[p]).wait()
    pltpu.make_async_copy(h_out.at[pl.ds(0, C1)],
                          bv.at[pl.ds(p * C1, C1)], sems[p]).wait()


def _p1_wait_wb(p, h_out, av, sems):
    pltpu.make_async_copy(h_out.at[pl.ds(0, C1)],
                          av.at[pl.ds(p * C1, C1)], sems[2 + p]).wait()


def _pass1_body(a_hbm, b_hbm, ti_hbm, si_hbm, h_hbm, mom_hbm,
                tiv, siv, av, bv, momv, sg0, sg1, sw0, sw1):
    c = lax.axis_index("c")
    s = lax.axis_index("s")
    e_pc = EPAD // NS
    nchunks = e_pc // C1
    a_tab = a_hbm.at[c]
    b_tab = b_hbm.at[c]
    h_out = h_hbm.at[c]
    row0 = s * (e_pc // KI1)
    sems = (sg0, sg1, sw0, sw1)

    _p1_issue(0, 0, a_tab, b_tab, ti_hbm, si_hbm, tiv, siv, av, bv,
              sems, row0)

    def step(jj, p, carry):
        np_ = 1 - p

        @pl.when(jj + 1 < nchunks)
        def _():
            @pl.when(jj >= 1)
            def _():
                _p1_wait_wb(np_, h_out, av, sems)
            _p1_issue(jj + 1, np_, a_tab, b_tab, ti_hbm, si_hbm,
                      tiv, siv, av, bv, sems, row0)

        _p1_wait_gather(p, h_out, av, bv, sems)

        @plsc.parallel_loop(0, C1, 1, unroll=8, carry=carry)
        def carry2(i, cr):
            t0, t1, p0, p1 = cr
            h0 = av[p * C1 + i, pl.ds(0, 16)] + bv[p * C1 + i, pl.ds(0, 16)]
            av[p * C1 + i, pl.ds(0, 16)] = h0
            h1 = (av[p * C1 + i, pl.ds(16, 16)]
                  + bv[p * C1 + i, pl.ds(16, 16)])
            av[p * C1 + i, pl.ds(16, 16)] = h1
            return (t0 + h0, t1 + h1, p0 + h0 * h0, p1 + h1 * h1)

        pltpu.async_copy(av.at[pl.ds(p * C1, C1)],
                         h_out.at[pl.ds(s * e_pc + jj * C1, C1)],
                         sems[2 + p])
        return carry2

    def pair(q, carry):
        carry = step(2 * q, 0, carry)
        carry = step(2 * q + 1, 1, carry)
        return carry

    z = jnp.zeros((16,), jnp.float32)
    s0, s1, q0, q1 = lax.fori_loop(0, nchunks // 2, pair, (z, z, z, z))
    _p1_wait_wb(0, h_out, av, sems)
    _p1_wait_wb(1, h_out, av, sems)
    momv[0, pl.ds(0, 16)] = s0
    momv[0, pl.ds(16, 16)] = s1
    momv[1, pl.ds(0, 16)] = q0
    momv[1, pl.ds(16, 16)] = q1
    pltpu.sync_copy(momv, mom_hbm.at[c].at[s])


def _pass1(a_tab, b_tab, ti, si):
    mesh = plsc.VectorSubcoreMesh(
        core_axis_name="c", subcore_axis_name="s",
        num_cores=NC, num_subcores=NS)
    return pl.kernel(
        _pass1_body,
        compiler_params=pltpu.CompilerParams(use_tc_tiling_on_sc=False),
        out_type=(
            jax.ShapeDtypeStruct((NC, EPAD, HH), jnp.float32),
            jax.ShapeDtypeStruct((NC, NS, 2, HH), jnp.float32),
        ),
        mesh=mesh,
        scratch_types=[
            pltpu.VMEM((2 * NS1, KI1), jnp.int32),
            pltpu.VMEM((2 * NS1, KI1), jnp.int32),
            pltpu.VMEM((2 * C1, HH), jnp.float32),
            pltpu.VMEM((2 * C1, HH), jnp.float32),
            pltpu.VMEM((2, HH), jnp.float32),
            pltpu.SemaphoreType.DMA,
            pltpu.SemaphoreType.DMA,
            pltpu.SemaphoreType.DMA,
            pltpu.SemaphoreType.DMA,
        ],
    )(a_tab, b_tab, ti, si)


def _mid_body(m_ref, g_ref, be_ref, e_ref, out_ref):
    e = e_ref[0, 0]
    m = m_ref[...]
    tot = jnp.sum(m, axis=1)
    mu = tot[:, 0, :] / e
    var = tot[:, 1, :] / e - mu * mu
    scale = g_ref[...] * jax.lax.rsqrt(var + 1e-5)
    shift = be_ref[...] - mu * scale
    out_ref[:, 0, :] = scale
    out_ref[:, 1, :] = shift


def _mid(mom, g, be, e_total):
    e = jnp.full((1, 1), float(e_total), jnp.float32)
    ss = jax.ShapeDtypeStruct((NC, 2, HH), jnp.float32)
    return pl.pallas_call(_mid_body, out_shape=ss)(mom, g, be, e)


C2 = 256
NS2 = C2 // KI2


def _p2_issue(jj, p, h_in, ti_hbm, tiv, yv, sems, row0, base):
    pltpu.sync_copy(ti_hbm.at[pl.ds(row0 + jj * NS2, NS2)],
                    tiv.at[pl.ds(p * NS2, NS2)])
    pltpu.async_copy(h_in.at[pl.ds(base + jj * C2, C2)],
                     yv.at[pl.ds(p * C2, C2)], sems[p])


def _p2_wait_h(p, h_in, yv, sems):
    pltpu.make_async_copy(h_in.at[pl.ds(0, C2)],
                          yv.at[pl.ds(p * C2, C2)], sems[p]).wait()


def _p2_wait_scat(p, shared, yv, sems):
    pltpu.make_async_copy(yv.at[pl.ds(0, C2)],
                          shared.at[pl.ds(0, C2)], sems[2 + p]).wait()


def _pass2_body(h_hbm, ti_hbm, ss_hbm, agg_hbm,
                shared, tiv, yv, ssv, sg0, sg1, ss0, ss1):
    c = lax.axis_index("c")
    s = lax.axis_index("s")
    e_pc = EPAD // NS
    nchunks = e_pc // C2
    rows_pt = NPAD // NS
    sems = (sg0, sg1, ss0, ss1)

    pltpu.sync_copy(ss_hbm.at[c], ssv)

    @plsc.parallel_loop(0, 2 * C2, 1, unroll=8)
    def _z(i):
        yv[i, pl.ds(0, 16)] = jnp.zeros((16,), jnp.float32)
        yv[i, pl.ds(16, 16)] = jnp.zeros((16,), jnp.float32)

    stripe0 = s * rows_pt
    for t in range(rows_pt // (2 * C2)):
        pltpu.sync_copy(yv, shared.at[pl.ds(stripe0 + t * 2 * C2, 2 * C2)])
    rem0 = (rows_pt // (2 * C2)) * 2 * C2
    pltpu.sync_copy(yv.at[pl.ds(0, rows_pt - rem0)],
                    shared.at[pl.ds(stripe0 + rem0, rows_pt - rem0)])
    plsc.subcore_barrier()

    sc0 = ssv[0, pl.ds(0, 16)]
    sc1 = ssv[0, pl.ds(16, 16)]
    sh0 = ssv[1, pl.ds(0, 16)]
    sh1 = ssv[1, pl.ds(16, 16)]
    h_in = h_hbm.at[c]
    row0 = s * (e_pc // KI2)
    base = s * e_pc

    _p2_issue(0, 0, h_in, ti_hbm, tiv, yv, sems, row0, base)

    def step(jj, p):
        np_ = 1 - p

        @pl.when(jj >= 1)
        def _():
            _p2_wait_scat(np_, shared, yv, sems)

        @pl.when(jj + 1 < nchunks)
        def _():
            _p2_issue(jj + 1, np_, h_in, ti_hbm, tiv, yv, sems, row0, base)

        _p2_wait_h(p, h_in, yv, sems)

        @plsc.parallel_loop(0, C2, 1, unroll=8)
        def _rows(i):
            y0 = yv[p * C2 + i, pl.ds(0, 16)] * sc0 + sh0
            yv[p * C2 + i, pl.ds(0, 16)] = jnp.maximum(y0, 0.0)
            y1 = yv[p * C2 + i, pl.ds(16, 16)] * sc1 + sh1
            yv[p * C2 + i, pl.ds(16, 16)] = jnp.maximum(y1, 0.0)

        for k in range(NS2):
            pltpu.async_copy(yv.at[pl.ds(p * C2 + k * KI2, KI2)],
                             shared.at[tiv.at[p * NS2 + k]],
                             sems[2 + p], add=True)

    def pair(q, _):
        step(2 * q, 0)
        step(2 * q + 1, 1)
        return 0

    lax.fori_loop(0, nchunks // 2, pair, 0)
    _p2_wait_scat(1, shared, yv, sems)
    plsc.subcore_barrier()
    pltpu.sync_copy(shared.at[pl.ds(s * rows_pt, rows_pt)],
                    agg_hbm.at[c].at[pl.ds(s * rows_pt, rows_pt)])


def _pass2(h, ti, ss):
    mesh = plsc.VectorSubcoreMesh(
        core_axis_name="c", subcore_axis_name="s",
        num_cores=NC, num_subcores=NS)
    return pl.kernel(
        _pass2_body,
        compiler_params=pltpu.CompilerParams(use_tc_tiling_on_sc=False),
        out_type=jax.ShapeDtypeStruct((NC, NPAD, HH), jnp.float32),
        mesh=mesh,
        scratch_types=[
            pltpu.VMEM_SHARED((NPAD, HH), jnp.float32),
            pltpu.VMEM((2 * NS2, KI2), jnp.int32),
            pltpu.VMEM((2 * C2, HH), jnp.float32),
            pltpu.VMEM((2, HH), jnp.float32),
            pltpu.SemaphoreType.DMA,
            pltpu.SemaphoreType.DMA,
            pltpu.SemaphoreType.DMA,
            pltpu.SemaphoreType.DMA,
        ],
    )(h, ti, ss)


BRD = 5000


def _affine(st_ref, g_ref, be_ref, nrows):
    s = st_ref[0:1, :]
    q = st_ref[1:2, :]
    mu = s / nrows
    var = q / nrows - mu * mu
    sc = g_ref[...] * jax.lax.rsqrt(var + 1e-5)
    sh = be_ref[...] - mu * sc
    return sc, sh


def _acc_stats(i, h, st_ref):
    @pl.when(i == 0)
    def _():
        st_ref[...] = jnp.zeros_like(st_ref)

    st_ref[0:1, :] += jnp.sum(h, axis=0, keepdims=True)
    st_ref[1:2, :] += jnp.sum(h * h, axis=0, keepdims=True)


def _t1_body(agg_ref, x_ref, eps_ref, w_ref, b_ref, h_ref, st_ref):
    i = pl.program_id(0)
    a = jnp.concatenate([agg_ref[0], agg_ref[1]], axis=-1)
    xin = a + (1.0 + eps_ref[0, 0]) * x_ref[...]
    h = jnp.dot(xin, w_ref[...], preferred_element_type=jnp.float32) + b_ref[...]
    h_ref[...] = h
    _acc_stats(i, h, st_ref)


def _t2_body(nrows, st_ref, g_ref, be_ref, h_ref, w_ref, b_ref,
             h2_ref, st2_ref):
    i = pl.program_id(0)
    sc, sh = _affine(st_ref, g_ref, be_ref, nrows)
    x1 = jnp.maximum(h_ref[...] * sc + sh, 0.0)
    h2 = jnp.dot(x1, w_ref[...], preferred_element_type=jnp.float32) + b_ref[...]
    h2_ref[...] = h2
    _acc_stats(i, h2, st2_ref)


def _t3_body(nrows, st_ref, g_ref, be_ref, h_ref, w_ref, p_ref):
    sc, sh = _affine(st_ref, g_ref, be_ref, nrows)
    x2 = jnp.maximum(h_ref[...] * sc + sh, 0.0)
    p_ref[...] = jnp.dot(x2, w_ref[...], preferred_element_type=jnp.float32)


def _t3c_body(nrows, st_ref, g_ref, be_ref, h_ref, w_ref, pu_ref, bco_ref,
              hco_ref, stco_ref):
    i = pl.program_id(0)
    sc, sh = _affine(st_ref, g_ref, be_ref, nrows)
    x2 = jnp.maximum(h_ref[...] * sc + sh, 0.0)
    hco = (pu_ref[...]
           + jnp.dot(x2, w_ref[...], preferred_element_type=jnp.float32)
           + bco_ref[...])
    hco_ref[...] = hco
    _acc_stats(i, hco, stco_ref)


def _t4_body(nrows, st_ref, g_ref, be_ref, h_ref, out_ref):
    sc, sh = _affine(st_ref, g_ref, be_ref, nrows)
    out_ref[...] = jnp.maximum(h_ref[...] * sc + sh, 0.0)


def _row_spec():
    return pl.BlockSpec((BRD, H), lambda i: (i, 0))


def _full_spec(shape):
    return pl.BlockSpec(shape, lambda i: tuple(0 for _ in shape))


def _dense_chain(agg_u, agg_b, x, p):
    n = x.shape[0]
    grid = n // BRD
    nf = float(n)
    hmat = jax.ShapeDtypeStruct((n, H), jnp.float32)
    stat = jax.ShapeDtypeStruct((2, H), jnp.float32)
    stat_spec = pl.BlockSpec((2, H), lambda i: (0, 0))

    def r2(v):
        return v.reshape(1, -1)

    def t1(agg, eps, w, b):
        return pl.pallas_call(
            _t1_body, grid=(grid,),
            in_specs=[pl.BlockSpec((NC, BRD, HH), lambda i: (0, i, 0)),
                      _row_spec(), _full_spec((1, 1)),
                      _full_spec((H, H)), _full_spec((1, H))],
            out_specs=[_row_spec(), stat_spec],
            out_shape=[hmat, stat],
        )(agg, x, eps.reshape(1, 1), w, r2(b))

    def t2(st, g, be, h, w, b):
        return pl.pallas_call(
            functools.partial(_t2_body, nf), grid=(grid,),
            in_specs=[stat_spec, _full_spec((1, H)), _full_spec((1, H)),
                      _row_spec(), _full_spec((H, H)), _full_spec((1, H))],
            out_specs=[_row_spec(), stat_spec],
            out_shape=[hmat, stat],
        )(st, r2(g), r2(be), h, w, r2(b))

    def t3(st, g, be, h, w):
        return pl.pallas_call(
            functools.partial(_t3_body, nf), grid=(grid,),
            in_specs=[stat_spec, _full_spec((1, H)), _full_spec((1, H)),
                      _row_spec(), _full_spec((H, H))],
            out_specs=_row_spec(),
            out_shape=hmat,
        )(st, r2(g), r2(be), h, w)

    def t3c(st, g, be, h, w, pu, bco):
        return pl.pallas_call(
            functools.partial(_t3c_body, nf), grid=(grid,),
            in_specs=[stat_spec, _full_spec((1, H)), _full_spec((1, H)),
                      _row_spec(), _full_spec((H, H)), _row_spec(),
                      _full_spec((1, H))],
            out_specs=[_row_spec(), stat_spec],
            out_shape=[hmat, stat],
        )(st, r2(g), r2(be), h, w, pu, r2(bco))

    def t4(st, g, be, h):
        return pl.pallas_call(
            functools.partial(_t4_body, nf), grid=(grid,),
            in_specs=[stat_spec, _full_spec((1, H)), _full_spec((1, H)),
                      _row_spec()],
            out_specs=_row_spec(),
            out_shape=hmat,
        )(st, r2(g), r2(be), h)

    h1u, s1u = t1(agg_u, p["eps1"], p["uu1_W"], p["uu1_b"])
    h2u, s2u = t2(s1u, p["uu1_g"], p["uu1_be"], h1u, p["uu2_W"], p["uu2_b"])
    pu = t3(s2u, p["uu2_g"], p["uu2_be"], h2u, p["co_W"][:H])

    h1b, s1b = t1(agg_b, p["eps2"], p["ub1_W"], p["ub1_b"])
    h2b, s2b = t2(s1b, p["ub1_g"], p["ub1_be"], h1b, p["ub2_W"], p["ub2_b"])
    hco, sco = t3c(s2b, p["ub2_g"], p["ub2_be"], h2b, p["co_W"][H:], pu,
                   p["co_b"])

    return t4(sco, p["co_g"], p["co_be"], hco)


def kernel(x, up_index, boundary_index, boundary_attr, params):
    p = params
    n = x.shape[0]
    e = up_index.shape[1]

    def r2(v):
        return v.reshape(1, -1)

    xp = jnp.pad(x, ((0, NPAD - n), (0, 0)))
    bap = jnp.pad(boundary_attr, ((0, NPAD - boundary_attr.shape[0]), (0, 0)))
    au, bu, ab, bb = _make_tables(
        xp, bap, p["mu_W"], r2(p["mu_b"]), p["mb_W"], r2(p["mb_b"]), n)

    pad_idx = n + (jnp.arange(EPAD - e, dtype=jnp.int32) % (NPAD - n))

    def prep(idx, ki):
        return jnp.concatenate([idx, pad_idx]).reshape(EPAD // ki, ki)

    ti_u1 = prep(up_index[0], KI1)
    si_u1 = prep(up_index[1], KI1)
    ti_b1 = prep(boundary_index[1], KI1)
    si_b1 = prep(boundary_index[0], KI1)
    ti_u2 = prep(up_index[0], KI2)
    ti_b2 = prep(boundary_index[1], KI2)

    h_u, mom_u = _pass1(au, bu, ti_u1, si_u1)
    ss_u = _mid(mom_u, p["mu_g"].reshape(NC, HH),
                p["mu_be"].reshape(NC, HH), e)
    agg_u = _pass2(h_u, ti_u2, ss_u)

    h_b, mom_b = _pass1(ab, bb, ti_b1, si_b1)
    ss_b = _mid(mom_b, p["mb_g"].reshape(NC, HH),
                p["mb_be"].reshape(NC, HH), e)
    agg_b = _pass2(h_b, ti_b2, ss_b)

    return _dense_chain(agg_u, agg_b, x, p)
